# baseline (speedup 1.0000x reference)
import dataclasses
import functools

import jax
import jax.numpy as jnp
from jax import lax
from jax.experimental import pallas as pl
from jax.experimental.pallas import tpu as pltpu
from jax.experimental.pallas import tpu_sc as plsc

N = 100000
E = 1600000
D_IN = 128
H = 32
L = 32

NC = 2
NT = 16
HALF = N // NC
IPART = 3120
ISZ = 3200
SCAN = E // NT
CHUNK = 2000
NVEC = CHUNK // 16
RING = 4096
SUP = 256
NISS = SUP // 128
CAP = 102400
CAPR = CAP // 128
ACC_ROWS = HALF + 48
HALFP = HALF + 48

_mesh = plsc.VectorSubcoreMesh(core_axis_name="c", subcore_axis_name="s")

_cp = pltpu.CompilerParams()
if "needs_layout_passes" in pltpu.CompilerParams.__dataclass_fields__:
    _cp = dataclasses.replace(_cp, needs_layout_passes=False)
if "use_tc_tiling_on_sc" in pltpu.CompilerParams.__dataclass_fields__:
    _cp = dataclasses.replace(_cp, use_tc_tiling_on_sc=False)


def _prepass_body(ei_hbm, srcs_hbm, dstl_hbm, cnts_hbm, parts_hbm,
                  ein_s, ein_d, ring_s, ring_d, deg, cntv):
    c = lax.axis_index("c")
    s = lax.axis_index("s")
    wid = c * NT + s
    base = c * HALF
    lane = jnp.arange(16, dtype=jnp.int32)
    ones = jnp.ones((16,), jnp.float32)

    @pl.loop(0, ACC_ROWS // 16)
    def _zero(i):
        deg[pl.ds(i * 16, 16)] = jnp.zeros((16,), jnp.float32)

    def chunk_body(k, carry):
        fill, flushed = carry
        off = s * SCAN + k * CHUNK
        pltpu.sync_copy(ei_hbm.at[pl.ds(off, CHUNK)], ein_s)
        pltpu.sync_copy(ei_hbm.at[pl.ds(E + off, CHUNK)], ein_d)

        def vec_body(i, fill):
            src16 = ein_s[pl.ds(i * 16, 16)]
            dst16 = ein_d[pl.ds(i * 16, 16)]
            mask = (dst16 >= base) & (dst16 < base + HALF)
            dstl16 = dst16 - base
            mi = jnp.where(mask, 1, 0).astype(jnp.int32)
            cs = plsc.cumsum(mi)
            pos = (fill + cs - 1) & (RING - 1)
            ridx = pos >> 7
            cidx = pos & 127
            plsc.store_scatter(ring_s, [ridx, cidx], src16, mask=mask)
            plsc.store_scatter(ring_d, [ridx, cidx], dstl16, mask=mask)
            plsc.addupdate_scatter(deg, [dstl16], ones, mask=mask)
            return fill + jnp.max(cs, initial=0)

        fill = lax.fori_loop(0, NVEC, vec_body, fill)

        def flush():
            r0 = pl.multiple_of((flushed >> 7) & (RING // 128 - 1), 16)
            h0 = pl.multiple_of(flushed >> 7, 8)
            pltpu.sync_copy(ring_s.at[pl.ds(r0, 16)],
                            srcs_hbm.at[wid, pl.ds(h0, 16)])
            pltpu.sync_copy(ring_d.at[pl.ds(r0, 16)],
                            dstl_hbm.at[wid, pl.ds(h0, 16)])

        do_flush = fill - flushed >= 2048
        pl.when(do_flush)(flush)
        flushed = jnp.where(do_flush, flushed + 2048, flushed)
        return fill, flushed

    fill, flushed = lax.fori_loop(0, SCAN // CHUNK, chunk_body,
                                  (jnp.int32(0), jnp.int32(0)))

    rem = fill - flushed
    padrem = (rem + 2 * SUP - 1) & ~(2 * SUP - 1)
    npad = padrem - rem

    def pad_body(i, _):
        pv = fill + i * 16 + lane
        mask = pv < flushed + padrem
        p = pv
        pos = p & (RING - 1)
        ridx = pos >> 7
        cidx = pos & 127
        plsc.store_scatter(ring_s, [ridx, cidx], jnp.zeros((16,), jnp.int32),
                           mask=mask)
        plsc.store_scatter(ring_d, [ridx, cidx], HALF + lane, mask=mask)
        return 0

    lax.fori_loop(0, (npad + 15) // 16, pad_body, 0)

    def final_flush(nrows):
        def go():
            r0 = pl.multiple_of((flushed >> 7) & (RING // 128 - 1), 8)
            h0 = pl.multiple_of(flushed >> 7, 8)
            pltpu.sync_copy(ring_s.at[pl.ds(r0, nrows)],
                            srcs_hbm.at[wid, pl.ds(h0, nrows)])
            pltpu.sync_copy(ring_d.at[pl.ds(r0, nrows)],
                            dstl_hbm.at[wid, pl.ds(h0, nrows)])
        return go

    pl.when(padrem == 512)(final_flush(4))
    pl.when(padrem == 1024)(final_flush(8))
    pl.when(padrem == 1536)(final_flush(12))
    pl.when(padrem == 2048)(final_flush(16))

    nsup = (flushed + padrem) // SUP
    cntv[...] = jnp.broadcast_to(nsup, (16,)).astype(jnp.int32)
    pltpu.sync_copy(cntv, cnts_hbm.at[wid])
    pltpu.sync_copy(deg, parts_hbm.at[c, s])


_prepass = pl.kernel(
    _prepass_body,
    out_type=(
        jax.ShapeDtypeStruct((NC * NT, CAPR, 128), jnp.int32),
        jax.ShapeDtypeStruct((NC * NT, CAPR, 128), jnp.int32),
        jax.ShapeDtypeStruct((NC * NT, 16), jnp.int32),
        jax.ShapeDtypeStruct((NC, NT, HALFP), jnp.float32),
    ),
    mesh=_mesh,
    scratch_types=[
        pltpu.VMEM((CHUNK,), jnp.int32),
        pltpu.VMEM((CHUNK,), jnp.int32),
        pltpu.VMEM((RING // 128, 128), jnp.int32),
        pltpu.VMEM((RING // 128, 128), jnp.int32),
        pltpu.VMEM((ACC_ROWS,), jnp.float32),
        pltpu.VMEM((16,), jnp.int32),
    ],
    compiler_params=_cp,
)


def _propagate_body(p_hbm, srcs_hbm, dstl_hbm, cnts_hbm, q_hbm,
                    sbuf0, sbuf1, dbuf0, dbuf1, rows0, rows1, cntv, acc,
                    gsem0, gsem1, ssem0, ssem1):
    c = lax.axis_index("c")
    s = lax.axis_index("s")
    wid = c * NT + s
    start = jnp.minimum(s * IPART, HALF - ISZ)
    row0 = c * HALF + start

    pltpu.sync_copy(cnts_hbm.at[wid], cntv)
    nsup = jnp.max(cntv[...], initial=0)

    pltpu.sync_copy(p_hbm.at[pl.ds(row0, ISZ)], acc.at[pl.ds(start, ISZ)])
    plsc.subcore_barrier()

    sb = (sbuf0, sbuf1)
    db = (dbuf0, dbuf1)
    rows = (rows0, rows1)
    gsem = (gsem0, gsem1)
    ssem = (ssem0, ssem1)

    def wait_gathers(p):
        for k in range(NISS):
            pltpu.make_async_copy(p_hbm.at[sb[p].at[k]],
                                  rows[p].at[pl.ds(k * 128, 128)],
                                  gsem[p]).wait()

    def wait_scatters(p):
        for k in range(NISS):
            pltpu.make_async_copy(rows[p].at[pl.ds(k * 128, 128)],
                                  acc.at[db[p].at[k]], ssem[p]).wait()

    def pair_body(j2, _):
        for p in (0, 1):
            q = 1 - p
            jg = 2 * j2 + p
            js = jg - 1

            pl.when(jg >= 2)(lambda p=p: wait_scatters(p))

            def scatter(q=q):
                wait_gathers(q)
                for k in range(NISS):
                    pltpu.async_copy(rows[q].at[pl.ds(k * 128, 128)],
                                     acc.at[db[q].at[k]], ssem[q], add=True)

            pl.when((js >= 0) & (js < nsup))(scatter)

            def gather(p=p, jg=jg):
                pltpu.sync_copy(srcs_hbm.at[wid, pl.ds(jg * NISS, NISS)],
                                sb[p])
                pltpu.sync_copy(dstl_hbm.at[wid, pl.ds(jg * NISS, NISS)],
                                db[p])
                for k in range(NISS):
                    pltpu.async_copy(p_hbm.at[sb[p].at[k]],
                                     rows[p].at[pl.ds(k * 128, 128)],
                                     gsem[p])

            pl.when(jg < nsup)(gather)
        return 0

    lax.fori_loop(0, nsup // 2 + 1, pair_body, 0)
    plsc.subcore_barrier()
    pltpu.sync_copy(acc.at[pl.ds(start, ISZ)], q_hbm.at[pl.ds(row0, ISZ)])


_propagate = pl.kernel(
    _propagate_body,
    out_type=jax.ShapeDtypeStruct((N, H), jnp.float32),
    mesh=_mesh,
    scratch_types=[
        pltpu.VMEM((NISS, 128), jnp.int32),
        pltpu.VMEM((NISS, 128), jnp.int32),
        pltpu.VMEM((NISS, 128), jnp.int32),
        pltpu.VMEM((NISS, 128), jnp.int32),
        pltpu.VMEM((SUP, H), jnp.float32),
        pltpu.VMEM((SUP, H), jnp.float32),
        pltpu.VMEM((16,), jnp.int32),
        pltpu.VMEM_SHARED((ACC_ROWS, H), jnp.float32),
        pltpu.SemaphoreType.DMA,
        pltpu.SemaphoreType.DMA,
        pltpu.SemaphoreType.DMA,
        pltpu.SemaphoreType.DMA,
    ],
    compiler_params=_cp,
)


BLK = 1000
_HI = lax.Precision.DEFAULT


def _tc_deg_body(parts_ref, degt_ref):
    deg = jnp.sum(parts_ref[0], axis=0) + 1.0
    degt_ref[...] = deg[:HALF, None]


def _tc_deg(parts):
    return pl.pallas_call(
        _tc_deg_body,
        grid=(NC,),
        in_specs=[pl.BlockSpec((1, NT, HALFP), lambda c: (c, 0, 0))],
        out_specs=pl.BlockSpec((HALF, 1), lambda c: (c, 0)),
        out_shape=jax.ShapeDtypeStruct((N, 1), jnp.float32),
    )(parts)


def _tc_first_body(x_ref, w0_ref, b0_ref, degt_ref, p0_ref):
    srt = lax.rsqrt(degt_ref[...])
    h0 = jnp.dot(x_ref[...], w0_ref[...], precision=_HI) + b0_ref[...]
    p0_ref[...] = srt * h0


def _tc_first(x, w0, b0, degt):
    return pl.pallas_call(
        _tc_first_body,
        grid=(N // BLK,),
        in_specs=[
            pl.BlockSpec((BLK, D_IN), lambda i: (i, 0)),
            pl.BlockSpec((D_IN, H), lambda i: (0, 0)),
            pl.BlockSpec((1, H), lambda i: (0, 0)),
            pl.BlockSpec((BLK, 1), lambda i: (i, 0)),
        ],
        out_specs=pl.BlockSpec((BLK, H), lambda i: (i, 0)),
        out_shape=jax.ShapeDtypeStruct((N, H), jnp.float32),
    )(x, w0, b0.reshape(1, H), degt)


def _tc_mid_body(q_ref, w_ref, b_ref, degt_ref, p_ref):
    srt = lax.rsqrt(degt_ref[...])
    mm = jnp.dot(q_ref[...], w_ref[...], precision=_HI)
    p_ref[...] = jnp.maximum(srt * srt * mm + srt * b_ref[...], 0.0)


def _tc_mid(q, w, b, degt):
    return pl.pallas_call(
        _tc_mid_body,
        grid=(N // BLK,),
        in_specs=[
            pl.BlockSpec((BLK, H), lambda i: (i, 0)),
            pl.BlockSpec((H, H), lambda i: (0, 0)),
            pl.BlockSpec((1, H), lambda i: (0, 0)),
            pl.BlockSpec((BLK, 1), lambda i: (i, 0)),
        ],
        out_specs=pl.BlockSpec((BLK, H), lambda i: (i, 0)),
        out_shape=jax.ShapeDtypeStruct((N, H), jnp.float32),
    )(q, w, b.reshape(1, H), degt)


def _tc_last_body(q_ref, w_ref, b_ref, degt_ref, wo_ref, bo_ref, o_ref):
    srt = lax.rsqrt(degt_ref[...])
    mm = jnp.dot(q_ref[...], w_ref[...], precision=_HI)
    h = jnp.maximum(srt * mm + b_ref[...], 0.0)
    o_ref[...] = jnp.dot(h, wo_ref[...], precision=_HI) + bo_ref[...]


def _tc_last(q, w, b, degt, wout, bout):
    return pl.pallas_call(
        _tc_last_body,
        grid=(N // BLK,),
        in_specs=[
            pl.BlockSpec((BLK, H), lambda i: (i, 0)),
            pl.BlockSpec((H, H), lambda i: (0, 0)),
            pl.BlockSpec((1, H), lambda i: (0, 0)),
            pl.BlockSpec((BLK, 1), lambda i: (i, 0)),
            pl.BlockSpec((H, D_IN), lambda i: (0, 0)),
            pl.BlockSpec((1, D_IN), lambda i: (0, 0)),
        ],
        out_specs=pl.BlockSpec((BLK, D_IN), lambda i: (i, 0)),
        out_shape=jax.ShapeDtypeStruct((N, D_IN), jnp.float32),
    )(q, w, b.reshape(1, H), degt, wout, bout.reshape(1, D_IN))


def kernel(x, edge_index, W0, b0, Ws, bs, Wout, bout):
    srcs, dstl, cnts, parts = _prepass(edge_index.reshape(2 * E))
    degt = _tc_deg(parts)
    p = _tc_first(x, W0, b0, degt)
    for i in range(L - 1):
        q = _propagate(p, srcs, dstl, cnts)
        p = _tc_mid(q, Ws[i], bs[i], degt)
    q = _propagate(p, srcs, dstl, cnts)
    return _tc_last(q, Ws[L - 1], bs[L - 1], degt, Wout, bout)


# speedup vs baseline: 1.1301x; 1.1301x over previous
import dataclasses
import functools

import jax
import jax.numpy as jnp
from jax import lax
from jax.experimental import pallas as pl
from jax.experimental.pallas import tpu as pltpu
from jax.experimental.pallas import tpu_sc as plsc

N = 100000
E = 1600000
D_IN = 128
H = 32
L = 32

NC = 2
NT = 16
HALF = N // NC
IPART = 3120
ISZ = 3200
SCAN = E // NT
CHUNK = 2000
NVEC = CHUNK // 16
RING = 4096
SUP = 512
CAP = 102400
CAPR = CAP // 128
ACC_ROWS = HALF + 48
HALFP = HALF + 48

_mesh = plsc.VectorSubcoreMesh(core_axis_name="c", subcore_axis_name="s")

_cp = pltpu.CompilerParams()
if "needs_layout_passes" in pltpu.CompilerParams.__dataclass_fields__:
    _cp = dataclasses.replace(_cp, needs_layout_passes=False)
if "use_tc_tiling_on_sc" in pltpu.CompilerParams.__dataclass_fields__:
    _cp = dataclasses.replace(_cp, use_tc_tiling_on_sc=False)


def _prepass_body(ei_hbm, srcs_hbm, dstl_hbm, cnts_hbm, parts_hbm,
                  ein_s, ein_d, ring_s, ring_d, deg, cntv):
    c = lax.axis_index("c")
    s = lax.axis_index("s")
    wid = c * NT + s
    base = c * HALF
    lane = jnp.arange(16, dtype=jnp.int32)
    ones = jnp.ones((16,), jnp.float32)

    @pl.loop(0, ACC_ROWS // 16)
    def _zero(i):
        deg[pl.ds(i * 16, 16)] = jnp.zeros((16,), jnp.float32)

    def chunk_body(k, carry):
        fill, flushed = carry
        off = s * SCAN + k * CHUNK
        pltpu.sync_copy(ei_hbm.at[pl.ds(off, CHUNK)], ein_s)
        pltpu.sync_copy(ei_hbm.at[pl.ds(E + off, CHUNK)], ein_d)

        def vec_body(i, fill):
            src16 = ein_s[pl.ds(i * 16, 16)]
            dst16 = ein_d[pl.ds(i * 16, 16)]
            mask = (dst16 >= base) & (dst16 < base + HALF)
            dstl16 = dst16 - base
            mi = jnp.where(mask, 1, 0).astype(jnp.int32)
            cs = plsc.cumsum(mi)
            pos = (fill + cs - 1) & (RING - 1)
            plsc.store_scatter(ring_s, [pos], src16, mask=mask)
            plsc.store_scatter(ring_d, [pos], dstl16, mask=mask)
            plsc.addupdate_scatter(deg, [dstl16], ones, mask=mask)
            return fill + jnp.max(cs, initial=0)

        fill = lax.fori_loop(0, NVEC, vec_body, fill)

        def flush():
            r0 = pl.multiple_of(flushed & (RING - 1), 2048)
            h0 = pl.multiple_of(flushed, 2048)
            pltpu.sync_copy(ring_s.at[pl.ds(r0, 2048)],
                            srcs_hbm.at[wid, pl.ds(h0, 2048)])
            pltpu.sync_copy(ring_d.at[pl.ds(r0, 2048)],
                            dstl_hbm.at[wid, pl.ds(h0, 2048)])

        do_flush = fill - flushed >= 2048
        pl.when(do_flush)(flush)
        flushed = jnp.where(do_flush, flushed + 2048, flushed)
        return fill, flushed

    fill, flushed = lax.fori_loop(0, SCAN // CHUNK, chunk_body,
                                  (jnp.int32(0), jnp.int32(0)))

    rem = fill - flushed
    padrem = (rem + 2 * SUP - 1) & ~(2 * SUP - 1)
    npad = padrem - rem

    def pad_body(i, _):
        pv = fill + i * 16 + lane
        mask = pv < flushed + padrem
        pos = pv & (RING - 1)
        plsc.store_scatter(ring_s, [pos], jnp.zeros((16,), jnp.int32),
                           mask=mask)
        plsc.store_scatter(ring_d, [pos], HALF + lane, mask=mask)
        return 0

    lax.fori_loop(0, (npad + 15) // 16, pad_body, 0)

    def final_flush(nw):
        def go():
            r0 = pl.multiple_of(flushed & (RING - 1), 2048)
            h0 = pl.multiple_of(flushed, 2048)
            pltpu.sync_copy(ring_s.at[pl.ds(r0, nw)],
                            srcs_hbm.at[wid, pl.ds(h0, nw)])
            pltpu.sync_copy(ring_d.at[pl.ds(r0, nw)],
                            dstl_hbm.at[wid, pl.ds(h0, nw)])
        return go

    pl.when(padrem == 512)(final_flush(512))
    pl.when(padrem == 1024)(final_flush(1024))
    pl.when(padrem == 1536)(final_flush(1536))
    pl.when(padrem == 2048)(final_flush(2048))

    nsup = (flushed + padrem) // SUP
    cntv[...] = jnp.broadcast_to(nsup, (16,)).astype(jnp.int32)
    pltpu.sync_copy(cntv, cnts_hbm.at[wid])
    pltpu.sync_copy(deg, parts_hbm.at[c, s])


_prepass = pl.kernel(
    _prepass_body,
    out_type=(
        jax.ShapeDtypeStruct((NC * NT, CAP), jnp.int32),
        jax.ShapeDtypeStruct((NC * NT, CAP), jnp.int32),
        jax.ShapeDtypeStruct((NC * NT, 16), jnp.int32),
        jax.ShapeDtypeStruct((NC, NT, HALFP), jnp.float32),
    ),
    mesh=_mesh,
    scratch_types=[
        pltpu.VMEM((CHUNK,), jnp.int32),
        pltpu.VMEM((CHUNK,), jnp.int32),
        pltpu.VMEM((RING,), jnp.int32),
        pltpu.VMEM((RING,), jnp.int32),
        pltpu.VMEM((ACC_ROWS,), jnp.float32),
        pltpu.VMEM((16,), jnp.int32),
    ],
    compiler_params=_cp,
)


def _propagate_body(p_hbm, srcs_hbm, dstl_hbm, cnts_hbm, q_hbm,
                    sbuf0, sbuf1, dbuf0, dbuf1, rows0, rows1, cntv, acc,
                    gsem0, gsem1, ssem0, ssem1):
    c = lax.axis_index("c")
    s = lax.axis_index("s")
    wid = c * NT + s
    start = jnp.minimum(s * IPART, HALF - ISZ)
    row0 = c * HALF + start

    pltpu.sync_copy(cnts_hbm.at[wid], cntv)
    nsup = jnp.max(cntv[...], initial=0)

    pltpu.sync_copy(p_hbm.at[pl.ds(row0, ISZ)], acc.at[pl.ds(start, ISZ)])
    plsc.subcore_barrier()

    def sup_body(j, _):
        pltpu.sync_copy(srcs_hbm.at[wid, pl.ds(j * SUP, SUP)], sbuf0)
        pltpu.sync_copy(dstl_hbm.at[wid, pl.ds(j * SUP, SUP)], dbuf0)
        pltpu.async_copy(p_hbm.at[sbuf0], rows0, gsem0).wait()
        pltpu.async_copy(rows0, acc.at[dbuf0], ssem0, add=True).wait()
        return 0

    lax.fori_loop(0, nsup, sup_body, 0)
    plsc.subcore_barrier()
    pltpu.sync_copy(acc.at[pl.ds(start, ISZ)], q_hbm.at[pl.ds(row0, ISZ)])


_propagate = pl.kernel(
    _propagate_body,
    out_type=jax.ShapeDtypeStruct((N, H), jnp.float32),
    mesh=_mesh,
    scratch_types=[
        pltpu.VMEM((SUP,), jnp.int32),
        pltpu.VMEM((SUP,), jnp.int32),
        pltpu.VMEM((SUP,), jnp.int32),
        pltpu.VMEM((SUP,), jnp.int32),
        pltpu.VMEM((SUP, H), jnp.float32),
        pltpu.VMEM((8, H), jnp.float32),
        pltpu.VMEM((16,), jnp.int32),
        pltpu.VMEM_SHARED((ACC_ROWS, H), jnp.float32),
        pltpu.SemaphoreType.DMA,
        pltpu.SemaphoreType.DMA,
        pltpu.SemaphoreType.DMA,
        pltpu.SemaphoreType.DMA,
    ],
    compiler_params=_cp,
)


BLK = 1000
_HI = lax.Precision.DEFAULT


def _tc_deg_body(parts_ref, degt_ref):
    deg = jnp.sum(parts_ref[0], axis=0) + 1.0
    degt_ref[...] = deg[:HALF, None]


def _tc_deg(parts):
    return pl.pallas_call(
        _tc_deg_body,
        grid=(NC,),
        in_specs=[pl.BlockSpec((1, NT, HALFP), lambda c: (c, 0, 0))],
        out_specs=pl.BlockSpec((HALF, 1), lambda c: (c, 0)),
        out_shape=jax.ShapeDtypeStruct((N, 1), jnp.float32),
    )(parts)


def _tc_first_body(x_ref, w0_ref, b0_ref, degt_ref, p0_ref):
    srt = lax.rsqrt(degt_ref[...])
    h0 = jnp.dot(x_ref[...], w0_ref[...], precision=_HI) + b0_ref[...]
    p0_ref[...] = srt * h0


def _tc_first(x, w0, b0, degt):
    return pl.pallas_call(
        _tc_first_body,
        grid=(N // BLK,),
        in_specs=[
            pl.BlockSpec((BLK, D_IN), lambda i: (i, 0)),
            pl.BlockSpec((D_IN, H), lambda i: (0, 0)),
            pl.BlockSpec((1, H), lambda i: (0, 0)),
            pl.BlockSpec((BLK, 1), lambda i: (i, 0)),
        ],
        out_specs=pl.BlockSpec((BLK, H), lambda i: (i, 0)),
        out_shape=jax.ShapeDtypeStruct((N, H), jnp.float32),
    )(x, w0, b0.reshape(1, H), degt)


def _tc_mid_body(q_ref, w_ref, b_ref, degt_ref, p_ref):
    srt = lax.rsqrt(degt_ref[...])
    mm = jnp.dot(q_ref[...], w_ref[...], precision=_HI)
    p_ref[...] = jnp.maximum(srt * srt * mm + srt * b_ref[...], 0.0)


def _tc_mid(q, w, b, degt):
    return pl.pallas_call(
        _tc_mid_body,
        grid=(N // BLK,),
        in_specs=[
            pl.BlockSpec((BLK, H), lambda i: (i, 0)),
            pl.BlockSpec((H, H), lambda i: (0, 0)),
            pl.BlockSpec((1, H), lambda i: (0, 0)),
            pl.BlockSpec((BLK, 1), lambda i: (i, 0)),
        ],
        out_specs=pl.BlockSpec((BLK, H), lambda i: (i, 0)),
        out_shape=jax.ShapeDtypeStruct((N, H), jnp.float32),
    )(q, w, b.reshape(1, H), degt)


def _tc_last_body(q_ref, w_ref, b_ref, degt_ref, wo_ref, bo_ref, o_ref):
    srt = lax.rsqrt(degt_ref[...])
    mm = jnp.dot(q_ref[...], w_ref[...], precision=_HI)
    h = jnp.maximum(srt * mm + b_ref[...], 0.0)
    o_ref[...] = jnp.dot(h, wo_ref[...], precision=_HI) + bo_ref[...]


def _tc_last(q, w, b, degt, wout, bout):
    return pl.pallas_call(
        _tc_last_body,
        grid=(N // BLK,),
        in_specs=[
            pl.BlockSpec((BLK, H), lambda i: (i, 0)),
            pl.BlockSpec((H, H), lambda i: (0, 0)),
            pl.BlockSpec((1, H), lambda i: (0, 0)),
            pl.BlockSpec((BLK, 1), lambda i: (i, 0)),
            pl.BlockSpec((H, D_IN), lambda i: (0, 0)),
            pl.BlockSpec((1, D_IN), lambda i: (0, 0)),
        ],
        out_specs=pl.BlockSpec((BLK, D_IN), lambda i: (i, 0)),
        out_shape=jax.ShapeDtypeStruct((N, D_IN), jnp.float32),
    )(q, w, b.reshape(1, H), degt, wout, bout.reshape(1, D_IN))


def kernel(x, edge_index, W0, b0, Ws, bs, Wout, bout):
    srcs, dstl, cnts, parts = _prepass(edge_index.reshape(2 * E))
    degt = _tc_deg(parts)
    p = _tc_first(x, W0, b0, degt)
    for i in range(L - 1):
        q = _propagate(p, srcs, dstl, cnts)
        p = _tc_mid(q, Ws[i], bs[i], degt)
    q = _propagate(p, srcs, dstl, cnts)
    return _tc_last(q, Ws[L - 1], bs[L - 1], degt, Wout, bout)


# speedup vs baseline: 1.6297x; 1.4420x over previous
import dataclasses
import functools

import jax
import jax.numpy as jnp
from jax import lax
from jax.experimental import pallas as pl
from jax.experimental.pallas import tpu as pltpu
from jax.experimental.pallas import tpu_sc as plsc

N = 100000
E = 1600000
D_IN = 128
H = 32
L = 32

NC = 2
NT = 16
HALF = N // NC
IPART = 3120
ISZ = 3200
SCAN = E // NT
CHUNK = 2000
NVEC = CHUNK // 16
RING = 4096
SUP = 256
CAP = 102400
CAPR = CAP // 128
ACC_ROWS = HALF + 48
HALFP = HALF + 48

_mesh = plsc.VectorSubcoreMesh(core_axis_name="c", subcore_axis_name="s")

_cp = pltpu.CompilerParams()
if "needs_layout_passes" in pltpu.CompilerParams.__dataclass_fields__:
    _cp = dataclasses.replace(_cp, needs_layout_passes=False)
if "use_tc_tiling_on_sc" in pltpu.CompilerParams.__dataclass_fields__:
    _cp = dataclasses.replace(_cp, use_tc_tiling_on_sc=False)


def _prepass_body(ei_hbm, srcs_hbm, dstl_hbm, cnts_hbm, parts_hbm,
                  ein_s, ein_d, ring_s, ring_d, deg, cntv):
    c = lax.axis_index("c")
    s = lax.axis_index("s")
    wid = c * NT + s
    base = c * HALF
    lane = jnp.arange(16, dtype=jnp.int32)
    ones = jnp.ones((16,), jnp.float32)

    @pl.loop(0, ACC_ROWS // 16)
    def _zero(i):
        deg[pl.ds(i * 16, 16)] = jnp.zeros((16,), jnp.float32)

    def chunk_body(k, carry):
        fill, flushed = carry
        off = s * SCAN + k * CHUNK
        pltpu.sync_copy(ei_hbm.at[pl.ds(off, CHUNK)], ein_s)
        pltpu.sync_copy(ei_hbm.at[pl.ds(E + off, CHUNK)], ein_d)

        def vec_body(i, fill):
            src16 = ein_s[pl.ds(i * 16, 16)]
            dst16 = ein_d[pl.ds(i * 16, 16)]
            mask = (dst16 >= base) & (dst16 < base + HALF)
            dstl16 = dst16 - base
            mi = jnp.where(mask, 1, 0).astype(jnp.int32)
            cs = plsc.cumsum(mi)
            pos = (fill + cs - 1) & (RING - 1)
            plsc.store_scatter(ring_s, [pos], src16, mask=mask)
            plsc.store_scatter(ring_d, [pos], dstl16, mask=mask)
            plsc.addupdate_scatter(deg, [dstl16], ones, mask=mask)
            return fill + jnp.max(cs, initial=0)

        fill = lax.fori_loop(0, NVEC, vec_body, fill)

        def flush():
            r0 = pl.multiple_of(flushed & (RING - 1), 2048)
            h0 = pl.multiple_of(flushed, 2048)
            pltpu.sync_copy(ring_s.at[pl.ds(r0, 2048)],
                            srcs_hbm.at[wid, pl.ds(h0, 2048)])
            pltpu.sync_copy(ring_d.at[pl.ds(r0, 2048)],
                            dstl_hbm.at[wid, pl.ds(h0, 2048)])

        do_flush = fill - flushed >= 2048
        pl.when(do_flush)(flush)
        flushed = jnp.where(do_flush, flushed + 2048, flushed)
        return fill, flushed

    fill, flushed = lax.fori_loop(0, SCAN // CHUNK, chunk_body,
                                  (jnp.int32(0), jnp.int32(0)))

    rem = fill - flushed
    padrem = (rem + 2 * SUP - 1) & ~(2 * SUP - 1)
    npad = padrem - rem

    def pad_body(i, _):
        pv = fill + i * 16 + lane
        mask = pv < flushed + padrem
        pos = pv & (RING - 1)
        plsc.store_scatter(ring_s, [pos], jnp.zeros((16,), jnp.int32),
                           mask=mask)
        plsc.store_scatter(ring_d, [pos], HALF + lane, mask=mask)
        return 0

    lax.fori_loop(0, (npad + 15) // 16, pad_body, 0)

    def final_flush(nw):
        def go():
            r0 = pl.multiple_of(flushed & (RING - 1), 2048)
            h0 = pl.multiple_of(flushed, 2048)
            pltpu.sync_copy(ring_s.at[pl.ds(r0, nw)],
                            srcs_hbm.at[wid, pl.ds(h0, nw)])
            pltpu.sync_copy(ring_d.at[pl.ds(r0, nw)],
                            dstl_hbm.at[wid, pl.ds(h0, nw)])
        return go

    pl.when(padrem == 512)(final_flush(512))
    pl.when(padrem == 1024)(final_flush(1024))
    pl.when(padrem == 1536)(final_flush(1536))
    pl.when(padrem == 2048)(final_flush(2048))

    nsup = (flushed + padrem) // SUP
    cntv[...] = jnp.broadcast_to(nsup, (16,)).astype(jnp.int32)
    pltpu.sync_copy(cntv, cnts_hbm.at[wid])
    pltpu.sync_copy(deg, parts_hbm.at[c, s])


_prepass = pl.kernel(
    _prepass_body,
    out_type=(
        jax.ShapeDtypeStruct((NC * NT, CAP), jnp.int32),
        jax.ShapeDtypeStruct((NC * NT, CAP), jnp.int32),
        jax.ShapeDtypeStruct((NC * NT, 16), jnp.int32),
        jax.ShapeDtypeStruct((NC, NT, HALFP), jnp.float32),
    ),
    mesh=_mesh,
    scratch_types=[
        pltpu.VMEM((CHUNK,), jnp.int32),
        pltpu.VMEM((CHUNK,), jnp.int32),
        pltpu.VMEM((RING,), jnp.int32),
        pltpu.VMEM((RING,), jnp.int32),
        pltpu.VMEM((ACC_ROWS,), jnp.float32),
        pltpu.VMEM((16,), jnp.int32),
    ],
    compiler_params=_cp,
)


def _propagate_body(p_hbm, srcs_hbm, dstl_hbm, cnts_hbm, q_hbm,
                    sbuf0, sbuf1, sbuf2, dbuf0, dbuf1, dbuf2,
                    rows0, rows1, rows2, cntv, acc,
                    gsem0, gsem1, gsem2, ssem0, ssem1, ssem2,
                    isem0, isem1, isem2):
    c = lax.axis_index("c")
    s = lax.axis_index("s")
    wid = c * NT + s
    start = jnp.minimum(s * IPART, HALF - ISZ)
    row0 = c * HALF + start

    pltpu.sync_copy(cnts_hbm.at[wid], cntv)
    nsup = jnp.max(cntv[...], initial=0)

    pltpu.sync_copy(p_hbm.at[pl.ds(row0, ISZ)], acc.at[pl.ds(start, ISZ)])
    plsc.subcore_barrier()

    sb = (sbuf0, sbuf1, sbuf2)
    db = (dbuf0, dbuf1, dbuf2)
    rows = (rows0, rows1, rows2)
    gsem = (gsem0, gsem1, gsem2)
    ssem = (ssem0, ssem1, ssem2)
    isem = (isem0, isem1, isem2)

    def prefetch_idx(p, j):
        pltpu.async_copy(srcs_hbm.at[wid, pl.ds(j * SUP, SUP)], sb[p],
                         isem[p])
        pltpu.async_copy(dstl_hbm.at[wid, pl.ds(j * SUP, SUP)], db[p],
                         isem[p])

    def wait_idx(p, j):
        pltpu.make_async_copy(srcs_hbm.at[wid, pl.ds(j * SUP, SUP)], sb[p],
                              isem[p]).wait()
        pltpu.make_async_copy(dstl_hbm.at[wid, pl.ds(j * SUP, SUP)], db[p],
                              isem[p]).wait()

    pl.when(nsup > 0)(lambda: prefetch_idx(0, 0))

    def tri_body(j3, _):
        for p in (0, 1, 2):
            j = 3 * j3 + p
            js = j - 2
            p2 = (p + 1) % 3

            pl.when((j >= 3) & (j - 3 < nsup))(
                lambda p=p: pltpu.make_async_copy(
                    rows[p], acc.at[db[p]], ssem[p]).wait())

            def fire_gather(p=p, j=j):
                wait_idx(p, j)
                pltpu.async_copy(p_hbm.at[sb[p]], rows[p], gsem[p])

            pl.when(j < nsup)(fire_gather)

            def fire_scatter(p2=p2):
                pltpu.make_async_copy(p_hbm.at[sb[p2]], rows[p2],
                                      gsem[p2]).wait()
                pltpu.async_copy(rows[p2], acc.at[db[p2]], ssem[p2],
                                 add=True)

            pl.when((js >= 0) & (js < nsup))(fire_scatter)

            pl.when(j + 1 < nsup)(
                lambda p2=p2, j=j: prefetch_idx(p2, j + 1))
        return 0

    lax.fori_loop(0, nsup // 3 + 2, tri_body, 0)
    plsc.subcore_barrier()
    pltpu.sync_copy(acc.at[pl.ds(start, ISZ)], q_hbm.at[pl.ds(row0, ISZ)])


_propagate = pl.kernel(
    _propagate_body,
    out_type=jax.ShapeDtypeStruct((N, H), jnp.float32),
    mesh=_mesh,
    scratch_types=(
        [pltpu.VMEM((SUP,), jnp.int32)] * 6
        + [pltpu.VMEM((SUP, H), jnp.float32)] * 3
        + [pltpu.VMEM((16,), jnp.int32),
           pltpu.VMEM_SHARED((ACC_ROWS, H), jnp.float32)]
        + [pltpu.SemaphoreType.DMA] * 9
    ),
    compiler_params=_cp,
)


BLK = 1000
_HI = lax.Precision.DEFAULT


def _tc_deg_body(parts_ref, degt_ref):
    deg = jnp.sum(parts_ref[0], axis=0) + 1.0
    degt_ref[...] = deg[:HALF, None]


def _tc_deg(parts):
    return pl.pallas_call(
        _tc_deg_body,
        grid=(NC,),
        in_specs=[pl.BlockSpec((1, NT, HALFP), lambda c: (c, 0, 0))],
        out_specs=pl.BlockSpec((HALF, 1), lambda c: (c, 0)),
        out_shape=jax.ShapeDtypeStruct((N, 1), jnp.float32),
    )(parts)


def _tc_first_body(x_ref, w0_ref, b0_ref, degt_ref, p0_ref):
    srt = lax.rsqrt(degt_ref[...])
    h0 = jnp.dot(x_ref[...], w0_ref[...], precision=_HI) + b0_ref[...]
    p0_ref[...] = srt * h0


def _tc_first(x, w0, b0, degt):
    return pl.pallas_call(
        _tc_first_body,
        grid=(N // BLK,),
        in_specs=[
            pl.BlockSpec((BLK, D_IN), lambda i: (i, 0)),
            pl.BlockSpec((D_IN, H), lambda i: (0, 0)),
            pl.BlockSpec((1, H), lambda i: (0, 0)),
            pl.BlockSpec((BLK, 1), lambda i: (i, 0)),
        ],
        out_specs=pl.BlockSpec((BLK, H), lambda i: (i, 0)),
        out_shape=jax.ShapeDtypeStruct((N, H), jnp.float32),
    )(x, w0, b0.reshape(1, H), degt)


def _tc_mid_body(q_ref, w_ref, b_ref, degt_ref, p_ref):
    srt = lax.rsqrt(degt_ref[...])
    mm = jnp.dot(q_ref[...], w_ref[...], precision=_HI)
    p_ref[...] = jnp.maximum(srt * srt * mm + srt * b_ref[...], 0.0)


def _tc_mid(q, w, b, degt):
    return pl.pallas_call(
        _tc_mid_body,
        grid=(N // BLK,),
        in_specs=[
            pl.BlockSpec((BLK, H), lambda i: (i, 0)),
            pl.BlockSpec((H, H), lambda i: (0, 0)),
            pl.BlockSpec((1, H), lambda i: (0, 0)),
            pl.BlockSpec((BLK, 1), lambda i: (i, 0)),
        ],
        out_specs=pl.BlockSpec((BLK, H), lambda i: (i, 0)),
        out_shape=jax.ShapeDtypeStruct((N, H), jnp.float32),
    )(q, w, b.reshape(1, H), degt)


def _tc_last_body(q_ref, w_ref, b_ref, degt_ref, wo_ref, bo_ref, o_ref):
    srt = lax.rsqrt(degt_ref[...])
    mm = jnp.dot(q_ref[...], w_ref[...], precision=_HI)
    h = jnp.maximum(srt * mm + b_ref[...], 0.0)
    o_ref[...] = jnp.dot(h, wo_ref[...], precision=_HI) + bo_ref[...]


def _tc_last(q, w, b, degt, wout, bout):
    return pl.pallas_call(
        _tc_last_body,
        grid=(N // BLK,),
        in_specs=[
            pl.BlockSpec((BLK, H), lambda i: (i, 0)),
            pl.BlockSpec((H, H), lambda i: (0, 0)),
            pl.BlockSpec((1, H), lambda i: (0, 0)),
            pl.BlockSpec((BLK, 1), lambda i: (i, 0)),
            pl.BlockSpec((H, D_IN), lambda i: (0, 0)),
            pl.BlockSpec((1, D_IN), lambda i: (0, 0)),
        ],
        out_specs=pl.BlockSpec((BLK, D_IN), lambda i: (i, 0)),
        out_shape=jax.ShapeDtypeStruct((N, D_IN), jnp.float32),
    )(q, w, b.reshape(1, H), degt, wout, bout.reshape(1, D_IN))


def kernel(x, edge_index, W0, b0, Ws, bs, Wout, bout):
    srcs, dstl, cnts, parts = _prepass(edge_index.reshape(2 * E))
    degt = _tc_deg(parts)
    p = _tc_first(x, W0, b0, degt)
    for i in range(L - 1):
        q = _propagate(p, srcs, dstl, cnts)
        p = _tc_mid(q, Ws[i], bs[i], degt)
    q = _propagate(p, srcs, dstl, cnts)
    return _tc_last(q, Ws[L - 1], bs[L - 1], degt, Wout, bout)


# speedup vs baseline: 1.8381x; 1.1279x over previous
import dataclasses
import functools

import jax
import jax.numpy as jnp
from jax import lax
from jax.experimental import pallas as pl
from jax.experimental.pallas import tpu as pltpu
from jax.experimental.pallas import tpu_sc as plsc

N = 100000
E = 1600000
D_IN = 128
H = 32
L = 32

NC = 2
NT = 16
HALF = N // NC
IPART = 3120
ISZ = 3200
SCAN = E // NT
CHUNK = 2000
NVEC = CHUNK // 16
RING = 4096
SUP = 256
CAP = 102400
CAPR = CAP // 128
ACC_ROWS = HALF + 48
HALFP = HALF + 48

_mesh = plsc.VectorSubcoreMesh(core_axis_name="c", subcore_axis_name="s")

_cp = pltpu.CompilerParams()
if "needs_layout_passes" in pltpu.CompilerParams.__dataclass_fields__:
    _cp = dataclasses.replace(_cp, needs_layout_passes=False)
if "use_tc_tiling_on_sc" in pltpu.CompilerParams.__dataclass_fields__:
    _cp = dataclasses.replace(_cp, use_tc_tiling_on_sc=False)


def _prepass_body(ei_hbm, srcs_hbm, dstl_hbm, cnts_hbm, parts_hbm,
                  ein_s, ein_d, ring_s, ring_d, deg, cntv):
    c = lax.axis_index("c")
    s = lax.axis_index("s")
    wid = c * NT + s
    base = c * HALF
    lane = jnp.arange(16, dtype=jnp.int32)
    ones = jnp.ones((16,), jnp.float32)

    @pl.loop(0, ACC_ROWS // 16)
    def _zero(i):
        deg[pl.ds(i * 16, 16)] = jnp.zeros((16,), jnp.float32)

    def chunk_body(k, carry):
        fill, flushed = carry
        off = s * SCAN + k * CHUNK
        pltpu.sync_copy(ei_hbm.at[pl.ds(off, CHUNK)], ein_s)
        pltpu.sync_copy(ei_hbm.at[pl.ds(E + off, CHUNK)], ein_d)

        def vec_body(i, fill):
            src16 = ein_s[pl.ds(i * 16, 16)]
            dst16 = ein_d[pl.ds(i * 16, 16)]
            mask = (dst16 >= base) & (dst16 < base + HALF)
            dstl16 = dst16 - base
            mi = jnp.where(mask, 1, 0).astype(jnp.int32)
            cs = plsc.cumsum(mi)
            pos = (fill + cs - 1) & (RING - 1)
            plsc.store_scatter(ring_s, [pos], src16, mask=mask)
            plsc.store_scatter(ring_d, [pos], dstl16, mask=mask)
            plsc.addupdate_scatter(deg, [dstl16], ones, mask=mask)
            return fill + jnp.max(cs, initial=0)

        fill = lax.fori_loop(0, NVEC, vec_body, fill)

        def flush():
            r0 = pl.multiple_of(flushed & (RING - 1), 2048)
            h0 = pl.multiple_of(flushed, 2048)
            pltpu.sync_copy(ring_s.at[pl.ds(r0, 2048)],
                            srcs_hbm.at[wid, pl.ds(h0, 2048)])
            pltpu.sync_copy(ring_d.at[pl.ds(r0, 2048)],
                            dstl_hbm.at[wid, pl.ds(h0, 2048)])

        do_flush = fill - flushed >= 2048
        pl.when(do_flush)(flush)
        flushed = jnp.where(do_flush, flushed + 2048, flushed)
        return fill, flushed

    fill, flushed = lax.fori_loop(0, SCAN // CHUNK, chunk_body,
                                  (jnp.int32(0), jnp.int32(0)))

    rem = fill - flushed
    padrem = (rem + 2 * SUP - 1) & ~(2 * SUP - 1)
    npad = padrem - rem

    def pad_body(i, _):
        pv = fill + i * 16 + lane
        mask = pv < flushed + padrem
        pos = pv & (RING - 1)
        plsc.store_scatter(ring_s, [pos], jnp.zeros((16,), jnp.int32),
                           mask=mask)
        plsc.store_scatter(ring_d, [pos], HALF + lane, mask=mask)
        return 0

    lax.fori_loop(0, (npad + 15) // 16, pad_body, 0)

    def final_flush(nw):
        def go():
            r0 = pl.multiple_of(flushed & (RING - 1), 2048)
            h0 = pl.multiple_of(flushed, 2048)
            pltpu.sync_copy(ring_s.at[pl.ds(r0, nw)],
                            srcs_hbm.at[wid, pl.ds(h0, nw)])
            pltpu.sync_copy(ring_d.at[pl.ds(r0, nw)],
                            dstl_hbm.at[wid, pl.ds(h0, nw)])
        return go

    pl.when(padrem == 512)(final_flush(512))
    pl.when(padrem == 1024)(final_flush(1024))
    pl.when(padrem == 1536)(final_flush(1536))
    pl.when(padrem == 2048)(final_flush(2048))

    nsup = (flushed + padrem) // SUP
    cntv[...] = jnp.broadcast_to(nsup, (16,)).astype(jnp.int32)
    pltpu.sync_copy(cntv, cnts_hbm.at[wid])
    pltpu.sync_copy(deg, parts_hbm.at[c, s])


_prepass = pl.kernel(
    _prepass_body,
    out_type=(
        jax.ShapeDtypeStruct((NC * NT, CAP), jnp.int32),
        jax.ShapeDtypeStruct((NC * NT, CAP), jnp.int32),
        jax.ShapeDtypeStruct((NC * NT, 16), jnp.int32),
        jax.ShapeDtypeStruct((NC, NT, HALFP), jnp.float32),
    ),
    mesh=_mesh,
    scratch_types=[
        pltpu.VMEM((CHUNK,), jnp.int32),
        pltpu.VMEM((CHUNK,), jnp.int32),
        pltpu.VMEM((RING,), jnp.int32),
        pltpu.VMEM((RING,), jnp.int32),
        pltpu.VMEM((ACC_ROWS,), jnp.float32),
        pltpu.VMEM((16,), jnp.int32),
    ],
    compiler_params=_cp,
)


def _propagate_body(p_hbm, srcs_hbm, dstl_hbm, cnts_hbm, q_hbm,
                    sbuf0, sbuf1, sbuf2, dbuf0, dbuf1, dbuf2,
                    rows0, rows1, rows2, cntv, acc,
                    gsem0, gsem1, gsem2, ssem0, ssem1, ssem2,
                    isem0, isem1, isem2):
    c = lax.axis_index("c")
    s = lax.axis_index("s")
    wid = c * NT + s
    start = jnp.minimum(s * IPART, HALF - ISZ)
    row0 = c * HALF + start

    pltpu.sync_copy(cnts_hbm.at[wid], cntv)
    nsup = jnp.max(cntv[...], initial=0)

    pltpu.sync_copy(p_hbm.at[pl.ds(row0, ISZ)], acc.at[pl.ds(start, ISZ)])
    plsc.subcore_barrier()

    sb = (sbuf0, sbuf1, sbuf2)
    db = (dbuf0, dbuf1, dbuf2)
    rows = (rows0, rows1, rows2)
    gsem = (gsem0, gsem1, gsem2)
    ssem = (ssem0, ssem1, ssem2)
    isem = (isem0, isem1, isem2)

    def prefetch_idx(p, j):
        pltpu.async_copy(srcs_hbm.at[wid, pl.ds(j * SUP, SUP)], sb[p],
                         isem[p])
        pltpu.async_copy(dstl_hbm.at[wid, pl.ds(j * SUP, SUP)], db[p],
                         isem[p])

    def wait_idx(p, j):
        pltpu.make_async_copy(srcs_hbm.at[wid, pl.ds(j * SUP, SUP)], sb[p],
                              isem[p]).wait()
        pltpu.make_async_copy(dstl_hbm.at[wid, pl.ds(j * SUP, SUP)], db[p],
                              isem[p]).wait()

    pl.when(nsup > 0)(lambda: prefetch_idx(0, 0))

    def tri_body(j3, _):
        for p in (0, 1, 2):
            j = 3 * j3 + p
            js = j - 2
            p2 = (p + 1) % 3

            pl.when((j >= 3) & (j - 3 < nsup))(
                lambda p=p: pltpu.make_async_copy(
                    rows[p], acc.at[db[p]], ssem[p]).wait())

            def fire_gather(p=p, j=j):
                wait_idx(p, j)
                pltpu.async_copy(p_hbm.at[sb[p]], rows[p], gsem[p])

            pl.when(j < nsup)(fire_gather)

            def fire_scatter(p2=p2):
                pltpu.make_async_copy(p_hbm.at[sb[p2]], rows[p2],
                                      gsem[p2]).wait()
                pltpu.async_copy(rows[p2], acc.at[db[p2]], ssem[p2],
                                 add=True)

            pl.when((js >= 0) & (js < nsup))(fire_scatter)

            pl.when(j + 1 < nsup)(
                lambda p2=p2, j=j: prefetch_idx(p2, j + 1))
        return 0

    lax.fori_loop(0, nsup // 3 + 2, tri_body, 0)
    plsc.subcore_barrier()
    pltpu.sync_copy(acc.at[pl.ds(start, ISZ)], q_hbm.at[pl.ds(row0, ISZ)])


_propagate = pl.kernel(
    _propagate_body,
    out_type=jax.ShapeDtypeStruct((N, H), jnp.float32),
    mesh=_mesh,
    scratch_types=(
        [pltpu.VMEM((SUP,), jnp.int32)] * 6
        + [pltpu.VMEM((SUP, H), jnp.float32)] * 3
        + [pltpu.VMEM((16,), jnp.int32),
           pltpu.VMEM_SHARED((ACC_ROWS, H), jnp.float32)]
        + [pltpu.SemaphoreType.DMA] * 9
    ),
    compiler_params=_cp,
)


BLK = 4000
_HI = lax.Precision.DEFAULT


def _tc_deg_body(parts_ref, degt_ref):
    deg = jnp.sum(parts_ref[0], axis=0) + 1.0
    degt_ref[...] = deg[:HALF, None]


def _tc_deg(parts):
    return pl.pallas_call(
        _tc_deg_body,
        grid=(NC,),
        in_specs=[pl.BlockSpec((1, NT, HALFP), lambda c: (c, 0, 0))],
        out_specs=pl.BlockSpec((HALF, 1), lambda c: (c, 0)),
        out_shape=jax.ShapeDtypeStruct((N, 1), jnp.float32),
    )(parts)


def _tc_first_body(x_ref, w0_ref, b0_ref, degt_ref, p0_ref):
    srt = lax.rsqrt(degt_ref[...])
    h0 = jnp.dot(x_ref[...], w0_ref[...], precision=_HI) + b0_ref[...]
    p0_ref[...] = srt * h0


def _tc_first(x, w0, b0, degt):
    return pl.pallas_call(
        _tc_first_body,
        grid=(N // BLK,),
        in_specs=[
            pl.BlockSpec((BLK, D_IN), lambda i: (i, 0)),
            pl.BlockSpec((D_IN, H), lambda i: (0, 0)),
            pl.BlockSpec((1, H), lambda i: (0, 0)),
            pl.BlockSpec((BLK, 1), lambda i: (i, 0)),
        ],
        out_specs=pl.BlockSpec((BLK, H), lambda i: (i, 0)),
        out_shape=jax.ShapeDtypeStruct((N, H), jnp.float32),
    )(x, w0, b0.reshape(1, H), degt)


def _tc_mid_body(q_ref, w_ref, b_ref, degt_ref, p_ref):
    srt = lax.rsqrt(degt_ref[...])
    mm = jnp.dot(q_ref[...], w_ref[...], precision=_HI)
    p_ref[...] = jnp.maximum(srt * srt * mm + srt * b_ref[...], 0.0)


def _tc_mid(q, w, b, degt):
    return pl.pallas_call(
        _tc_mid_body,
        grid=(N // BLK,),
        in_specs=[
            pl.BlockSpec((BLK, H), lambda i: (i, 0)),
            pl.BlockSpec((H, H), lambda i: (0, 0)),
            pl.BlockSpec((1, H), lambda i: (0, 0)),
            pl.BlockSpec((BLK, 1), lambda i: (i, 0)),
        ],
        out_specs=pl.BlockSpec((BLK, H), lambda i: (i, 0)),
        out_shape=jax.ShapeDtypeStruct((N, H), jnp.float32),
    )(q, w, b.reshape(1, H), degt)


def _tc_last_body(q_ref, w_ref, b_ref, degt_ref, wo_ref, bo_ref, o_ref):
    srt = lax.rsqrt(degt_ref[...])
    mm = jnp.dot(q_ref[...], w_ref[...], precision=_HI)
    h = jnp.maximum(srt * mm + b_ref[...], 0.0)
    o_ref[...] = jnp.dot(h, wo_ref[...], precision=_HI) + bo_ref[...]


def _tc_last(q, w, b, degt, wout, bout):
    return pl.pallas_call(
        _tc_last_body,
        grid=(N // BLK,),
        in_specs=[
            pl.BlockSpec((BLK, H), lambda i: (i, 0)),
            pl.BlockSpec((H, H), lambda i: (0, 0)),
            pl.BlockSpec((1, H), lambda i: (0, 0)),
            pl.BlockSpec((BLK, 1), lambda i: (i, 0)),
            pl.BlockSpec((H, D_IN), lambda i: (0, 0)),
            pl.BlockSpec((1, D_IN), lambda i: (0, 0)),
        ],
        out_specs=pl.BlockSpec((BLK, D_IN), lambda i: (i, 0)),
        out_shape=jax.ShapeDtypeStruct((N, D_IN), jnp.float32),
    )(q, w, b.reshape(1, H), degt, wout, bout.reshape(1, D_IN))


def kernel(x, edge_index, W0, b0, Ws, bs, Wout, bout):
    srcs, dstl, cnts, parts = _prepass(edge_index.reshape(2 * E))
    degt = _tc_deg(parts)
    p = _tc_first(x, W0, b0, degt)
    for i in range(L - 1):
        q = _propagate(p, srcs, dstl, cnts)
        p = _tc_mid(q, Ws[i], bs[i], degt)
    q = _propagate(p, srcs, dstl, cnts)
    return _tc_last(q, Ws[L - 1], bs[L - 1], degt, Wout, bout)


# speedup vs baseline: 1.8596x; 1.0117x over previous
import dataclasses
import functools

import jax
import jax.numpy as jnp
from jax import lax
from jax.experimental import pallas as pl
from jax.experimental.pallas import tpu as pltpu
from jax.experimental.pallas import tpu_sc as plsc

N = 100000
E = 1600000
D_IN = 128
H = 32
L = 32

NC = 2
NT = 16
HALF = N // NC
IPART = 3120
ISZ = 3200
SCAN = E // NT
CHUNK = 2000
NVEC = CHUNK // 16
RING = 4096
SUP = 256
CAP = 102400
CAPR = CAP // 128
ACC_ROWS = HALF + 48
HALFP = HALF + 48

_mesh = plsc.VectorSubcoreMesh(core_axis_name="c", subcore_axis_name="s")

_cp = pltpu.CompilerParams()
if "needs_layout_passes" in pltpu.CompilerParams.__dataclass_fields__:
    _cp = dataclasses.replace(_cp, needs_layout_passes=False)
if "use_tc_tiling_on_sc" in pltpu.CompilerParams.__dataclass_fields__:
    _cp = dataclasses.replace(_cp, use_tc_tiling_on_sc=False)


def _prepass_body(ei_hbm, srcs_hbm, dstl_hbm, cnts_hbm, parts_hbm,
                  ein_s, ein_d, ring_s, ring_d, deg, cntv):
    c = lax.axis_index("c")
    s = lax.axis_index("s")
    wid = c * NT + s
    base = c * HALF
    lane = jnp.arange(16, dtype=jnp.int32)
    ones = jnp.ones((16,), jnp.float32)

    @pl.loop(0, ACC_ROWS // 16)
    def _zero(i):
        deg[pl.ds(i * 16, 16)] = jnp.zeros((16,), jnp.float32)

    def chunk_body(k, carry):
        fill, flushed = carry
        off = s * SCAN + k * CHUNK
        pltpu.sync_copy(ei_hbm.at[pl.ds(off, CHUNK)], ein_s)
        pltpu.sync_copy(ei_hbm.at[pl.ds(E + off, CHUNK)], ein_d)

        def vec_body(i, fill):
            src16 = ein_s[pl.ds(i * 16, 16)]
            dst16 = ein_d[pl.ds(i * 16, 16)]
            mask = (dst16 >= base) & (dst16 < base + HALF)
            dstl16 = dst16 - base
            mi = jnp.where(mask, 1, 0).astype(jnp.int32)
            cs = plsc.cumsum(mi)
            pos = (fill + cs - 1) & (RING - 1)
            plsc.store_scatter(ring_s, [pos], src16, mask=mask)
            plsc.store_scatter(ring_d, [pos], dstl16, mask=mask)
            plsc.addupdate_scatter(deg, [dstl16], ones, mask=mask)
            return fill + jnp.max(cs, initial=0)

        fill = lax.fori_loop(0, NVEC, vec_body, fill)

        def flush():
            r0 = pl.multiple_of(flushed & (RING - 1), 2048)
            h0 = pl.multiple_of(flushed, 2048)
            pltpu.sync_copy(ring_s.at[pl.ds(r0, 2048)],
                            srcs_hbm.at[wid, pl.ds(h0, 2048)])
            pltpu.sync_copy(ring_d.at[pl.ds(r0, 2048)],
                            dstl_hbm.at[wid, pl.ds(h0, 2048)])

        do_flush = fill - flushed >= 2048
        pl.when(do_flush)(flush)
        flushed = jnp.where(do_flush, flushed + 2048, flushed)
        return fill, flushed

    fill, flushed = lax.fori_loop(0, SCAN // CHUNK, chunk_body,
                                  (jnp.int32(0), jnp.int32(0)))

    rem = fill - flushed
    padrem = (rem + 2 * SUP - 1) & ~(2 * SUP - 1)
    npad = padrem - rem

    def pad_body(i, _):
        pv = fill + i * 16 + lane
        mask = pv < flushed + padrem
        pos = pv & (RING - 1)
        plsc.store_scatter(ring_s, [pos], jnp.zeros((16,), jnp.int32),
                           mask=mask)
        plsc.store_scatter(ring_d, [pos], HALF + lane, mask=mask)
        return 0

    lax.fori_loop(0, (npad + 15) // 16, pad_body, 0)

    def final_flush(nw):
        def go():
            r0 = pl.multiple_of(flushed & (RING - 1), 2048)
            h0 = pl.multiple_of(flushed, 2048)
            pltpu.sync_copy(ring_s.at[pl.ds(r0, nw)],
                            srcs_hbm.at[wid, pl.ds(h0, nw)])
            pltpu.sync_copy(ring_d.at[pl.ds(r0, nw)],
                            dstl_hbm.at[wid, pl.ds(h0, nw)])
        return go

    pl.when(padrem == 512)(final_flush(512))
    pl.when(padrem == 1024)(final_flush(1024))
    pl.when(padrem == 1536)(final_flush(1536))
    pl.when(padrem == 2048)(final_flush(2048))

    nsup = (flushed + padrem) // SUP
    cntv[...] = jnp.broadcast_to(nsup, (16,)).astype(jnp.int32)
    pltpu.sync_copy(cntv, cnts_hbm.at[wid])
    pltpu.sync_copy(deg, parts_hbm.at[c, s])


_prepass = pl.kernel(
    _prepass_body,
    out_type=(
        jax.ShapeDtypeStruct((NC * NT, CAP), jnp.int32),
        jax.ShapeDtypeStruct((NC * NT, CAP), jnp.int32),
        jax.ShapeDtypeStruct((NC * NT, 16), jnp.int32),
        jax.ShapeDtypeStruct((NC, NT, HALFP), jnp.float32),
    ),
    mesh=_mesh,
    scratch_types=[
        pltpu.VMEM((CHUNK,), jnp.int32),
        pltpu.VMEM((CHUNK,), jnp.int32),
        pltpu.VMEM((RING,), jnp.int32),
        pltpu.VMEM((RING,), jnp.int32),
        pltpu.VMEM((ACC_ROWS,), jnp.float32),
        pltpu.VMEM((16,), jnp.int32),
    ],
    compiler_params=_cp,
)


def _propagate_body(p_hbm, srcs_hbm, dstl_hbm, cnts_hbm, q_hbm,
                    sbuf0, sbuf1, sbuf2, dbuf0, dbuf1, dbuf2,
                    rows0, rows1, rows2, cntv, acc,
                    gsem0, gsem1, gsem2, ssem0, ssem1, ssem2,
                    isem0, isem1, isem2):
    c = lax.axis_index("c")
    s = lax.axis_index("s")
    wid = c * NT + s
    start = jnp.minimum(s * IPART, HALF - ISZ)
    row0 = c * HALF + start

    pltpu.sync_copy(cnts_hbm.at[wid], cntv)
    nsup = jnp.max(cntv[...], initial=0)

    pltpu.sync_copy(p_hbm.at[pl.ds(row0, ISZ)], acc.at[pl.ds(start, ISZ)])
    plsc.subcore_barrier()

    sb = (sbuf0, sbuf1, sbuf2)
    db = (dbuf0, dbuf1, dbuf2)
    rows = (rows0, rows1, rows2)
    gsem = (gsem0, gsem1, gsem2)
    ssem = (ssem0, ssem1, ssem2)
    isem = (isem0, isem1, isem2)

    def prefetch_idx(p, j):
        pltpu.async_copy(srcs_hbm.at[wid, pl.ds(j * SUP, SUP)], sb[p],
                         isem[p])
        pltpu.async_copy(dstl_hbm.at[wid, pl.ds(j * SUP, SUP)], db[p],
                         isem[p])

    def wait_idx(p, j):
        pltpu.make_async_copy(srcs_hbm.at[wid, pl.ds(j * SUP, SUP)], sb[p],
                              isem[p]).wait()
        pltpu.make_async_copy(dstl_hbm.at[wid, pl.ds(j * SUP, SUP)], db[p],
                              isem[p]).wait()

    pl.when(nsup > 0)(lambda: prefetch_idx(0, 0))

    def tri_body(j3, _):
        for p in (0, 1, 2):
            j = 3 * j3 + p
            js = j - 2
            p2 = (p + 1) % 3

            pl.when((j >= 3) & (j - 3 < nsup))(
                lambda p=p: pltpu.make_async_copy(
                    rows[p], acc.at[db[p]], ssem[p]).wait())

            def fire_gather(p=p, j=j):
                wait_idx(p, j)
                pltpu.async_copy(p_hbm.at[sb[p]], rows[p], gsem[p])

            pl.when(j < nsup)(fire_gather)

            def fire_scatter(p2=p2):
                pltpu.make_async_copy(p_hbm.at[sb[p2]], rows[p2],
                                      gsem[p2]).wait()
                pltpu.async_copy(rows[p2], acc.at[db[p2]], ssem[p2],
                                 add=True)

            pl.when((js >= 0) & (js < nsup))(fire_scatter)

            pl.when(j + 1 < nsup)(
                lambda p2=p2, j=j: prefetch_idx(p2, j + 1))
        return 0

    lax.fori_loop(0, nsup // 3 + 2, tri_body, 0)
    plsc.subcore_barrier()
    pltpu.sync_copy(acc.at[pl.ds(start, ISZ)], q_hbm.at[pl.ds(row0, ISZ)])


_propagate = pl.kernel(
    _propagate_body,
    out_type=jax.ShapeDtypeStruct((N, H), jnp.float32),
    mesh=_mesh,
    scratch_types=(
        [pltpu.VMEM((SUP,), jnp.int32)] * 6
        + [pltpu.VMEM((SUP, H), jnp.float32)] * 3
        + [pltpu.VMEM((16,), jnp.int32),
           pltpu.VMEM_SHARED((ACC_ROWS, H), jnp.float32)]
        + [pltpu.SemaphoreType.DMA] * 9
    ),
    compiler_params=_cp,
)


BLK = 10000
_HI = lax.Precision.DEFAULT


def _tc_deg_body(parts_ref, degt_ref):
    deg = jnp.sum(parts_ref[0], axis=0) + 1.0
    degt_ref[...] = deg[:HALF, None]


def _tc_deg(parts):
    return pl.pallas_call(
        _tc_deg_body,
        grid=(NC,),
        in_specs=[pl.BlockSpec((1, NT, HALFP), lambda c: (c, 0, 0))],
        out_specs=pl.BlockSpec((HALF, 1), lambda c: (c, 0)),
        out_shape=jax.ShapeDtypeStruct((N, 1), jnp.float32),
    )(parts)


def _tc_first_body(x_ref, w0_ref, b0_ref, degt_ref, p0_ref):
    srt = lax.rsqrt(degt_ref[...])
    h0 = jnp.dot(x_ref[...], w0_ref[...], precision=_HI) + b0_ref[...]
    p0_ref[...] = srt * h0


def _tc_first(x, w0, b0, degt):
    return pl.pallas_call(
        _tc_first_body,
        grid=(N // BLK,),
        in_specs=[
            pl.BlockSpec((BLK, D_IN), lambda i: (i, 0)),
            pl.BlockSpec((D_IN, H), lambda i: (0, 0)),
            pl.BlockSpec((1, H), lambda i: (0, 0)),
            pl.BlockSpec((BLK, 1), lambda i: (i, 0)),
        ],
        out_specs=pl.BlockSpec((BLK, H), lambda i: (i, 0)),
        out_shape=jax.ShapeDtypeStruct((N, H), jnp.float32),
    )(x, w0, b0.reshape(1, H), degt)


def _tc_mid_body(q_ref, w_ref, b_ref, degt_ref, p_ref):
    srt = lax.rsqrt(degt_ref[...])
    mm = jnp.dot(q_ref[...], w_ref[...], precision=_HI)
    p_ref[...] = jnp.maximum(srt * srt * mm + srt * b_ref[...], 0.0)


def _tc_mid(q, w, b, degt):
    return pl.pallas_call(
        _tc_mid_body,
        grid=(N // BLK,),
        in_specs=[
            pl.BlockSpec((BLK, H), lambda i: (i, 0)),
            pl.BlockSpec((H, H), lambda i: (0, 0)),
            pl.BlockSpec((1, H), lambda i: (0, 0)),
            pl.BlockSpec((BLK, 1), lambda i: (i, 0)),
        ],
        out_specs=pl.BlockSpec((BLK, H), lambda i: (i, 0)),
        out_shape=jax.ShapeDtypeStruct((N, H), jnp.float32),
    )(q, w, b.reshape(1, H), degt)


def _tc_last_body(q_ref, w_ref, b_ref, degt_ref, wo_ref, bo_ref, o_ref):
    srt = lax.rsqrt(degt_ref[...])
    mm = jnp.dot(q_ref[...], w_ref[...], precision=_HI)
    h = jnp.maximum(srt * mm + b_ref[...], 0.0)
    o_ref[...] = jnp.dot(h, wo_ref[...], precision=_HI) + bo_ref[...]


def _tc_last(q, w, b, degt, wout, bout):
    return pl.pallas_call(
        _tc_last_body,
        grid=(N // BLK,),
        in_specs=[
            pl.BlockSpec((BLK, H), lambda i: (i, 0)),
            pl.BlockSpec((H, H), lambda i: (0, 0)),
            pl.BlockSpec((1, H), lambda i: (0, 0)),
            pl.BlockSpec((BLK, 1), lambda i: (i, 0)),
            pl.BlockSpec((H, D_IN), lambda i: (0, 0)),
            pl.BlockSpec((1, D_IN), lambda i: (0, 0)),
        ],
        out_specs=pl.BlockSpec((BLK, D_IN), lambda i: (i, 0)),
        out_shape=jax.ShapeDtypeStruct((N, D_IN), jnp.float32),
    )(q, w, b.reshape(1, H), degt, wout, bout.reshape(1, D_IN))


def kernel(x, edge_index, W0, b0, Ws, bs, Wout, bout):
    srcs, dstl, cnts, parts = _prepass(edge_index.reshape(2 * E))
    degt = _tc_deg(parts)
    p = _tc_first(x, W0, b0, degt)
    for i in range(L - 1):
        q = _propagate(p, srcs, dstl, cnts)
        p = _tc_mid(q, Ws[i], bs[i], degt)
    q = _propagate(p, srcs, dstl, cnts)
    return _tc_last(q, Ws[L - 1], bs[L - 1], degt, Wout, bout)
